# baseline (speedup 1.0000x reference)
import jax
import jax.numpy as jnp
from jax.experimental import pallas as pl
from jax.experimental.pallas import tpu as pltpu

N, E = 1_000_000, 256
BLOCK = 12_288


def _body(x_ref, w_ref, o_ref):
    x = x_ref[...]
    wr = w_ref[...]
    xx = x * x
    dims = (((1,), (1,)), ((), ()))
    s = (jax.lax.dot_general(wr[:, :128], x[:, :128], dims,
                             preferred_element_type=jnp.float32)
         + jax.lax.dot_general(wr[:, 128:], x[:, 128:], dims,
                               preferred_element_type=jnp.float32))
    num = (jax.lax.dot_general(wr[:, :128], xx[:, :128], dims,
                               preferred_element_type=jnp.float32)
           + jax.lax.dot_general(wr[:, 128:], xx[:, 128:], dims,
                                 preferred_element_type=jnp.float32))
    o_ref[...] = jnp.where(s == 0.0, 0.0, num / s).reshape(BLOCK)


def kernel(x, w):
    grid = (pl.cdiv(N, BLOCK),)
    out = pl.pallas_call(
        _body,
        grid=grid,
        in_specs=[
            pl.BlockSpec((BLOCK, E), lambda i: (i, 0)),
            pl.BlockSpec((1, E), lambda i: (0, 0)),
        ],
        out_specs=pl.BlockSpec((BLOCK,), lambda i: (i,)),
        out_shape=jax.ShapeDtypeStruct((N,), jnp.float32),
        compiler_params=pltpu.CompilerParams(
            dimension_semantics=("parallel",),
        ),
    )(x, w.reshape(1, E))
    return out


# speedup vs baseline: 1.0012x; 1.0012x over previous
import jax
import jax.numpy as jnp
from jax.experimental import pallas as pl
from jax.experimental.pallas import tpu as pltpu

N, E = 1_000_000, 256
BLOCK = 16_384
STEPS = 31


def _body(x_ref, w_hbm, o_ref, w_vmem, sem):
    j = pl.program_id(1)

    @pl.when(j == 0)
    def _():
        cp = pltpu.make_async_copy(w_hbm, w_vmem, sem)
        cp.start()
        cp.wait()

    x = x_ref[...]
    wr = w_vmem[...]
    xx = x * x
    dims = (((1,), (1,)), ((), ()))
    s = (jax.lax.dot_general(wr[:, :128], x[:, :128], dims,
                             preferred_element_type=jnp.float32)
         + jax.lax.dot_general(wr[:, 128:], x[:, 128:], dims,
                               preferred_element_type=jnp.float32))
    num = (jax.lax.dot_general(wr[:, :128], xx[:, :128], dims,
                               preferred_element_type=jnp.float32)
           + jax.lax.dot_general(wr[:, 128:], xx[:, 128:], dims,
                                 preferred_element_type=jnp.float32))
    o_ref[...] = jnp.where(s == 0.0, 0.0, num / s).reshape(BLOCK)


def kernel(x, w):
    out = pl.pallas_call(
        _body,
        grid=(2, STEPS),
        in_specs=[
            pl.BlockSpec((BLOCK, E), lambda c, j: (c * STEPS + j, 0)),
            pl.BlockSpec(memory_space=pl.ANY),
        ],
        out_specs=pl.BlockSpec((BLOCK,), lambda c, j: (c * STEPS + j,)),
        out_shape=jax.ShapeDtypeStruct((N,), jnp.float32),
        scratch_shapes=[
            pltpu.VMEM((1, E), jnp.float32),
            pltpu.SemaphoreType.DMA,
        ],
        compiler_params=pltpu.CompilerParams(
            dimension_semantics=("parallel", "arbitrary"),
        ),
    )(x, w.reshape(1, E))
    return out


# speedup vs baseline: 1.0595x; 1.0582x over previous
import jax
import jax.numpy as jnp
from jax.experimental import pallas as pl
from jax.experimental.pallas import tpu as pltpu

N, E = 1_000_000, 256
BLOCK = 20_480


def _body(x_ref, w_ref, o_ref):
    wr = w_ref[...]
    x_lo = x_ref[:, :128]
    x_hi = x_ref[:, 128:]
    dims = (((1,), (1,)), ((), ()))
    s = (jax.lax.dot_general(wr[:, :128], x_lo, dims,
                             preferred_element_type=jnp.float32)
         + jax.lax.dot_general(wr[:, 128:], x_hi, dims,
                               preferred_element_type=jnp.float32))
    num = (jax.lax.dot_general(wr[:, :128], x_lo * x_lo, dims,
                               preferred_element_type=jnp.float32)
           + jax.lax.dot_general(wr[:, 128:], x_hi * x_hi, dims,
                                 preferred_element_type=jnp.float32))
    o_ref[...] = jnp.where(s == 0.0, 0.0, num / s).reshape(BLOCK)


def kernel(x, w):
    grid = (pl.cdiv(N, BLOCK),)
    out = pl.pallas_call(
        _body,
        grid=grid,
        in_specs=[
            pl.BlockSpec((BLOCK, E), lambda i: (i, 0)),
            pl.BlockSpec((1, E), lambda i: (0, 0)),
        ],
        out_specs=pl.BlockSpec((BLOCK,), lambda i: (i,)),
        out_shape=jax.ShapeDtypeStruct((N,), jnp.float32),
        compiler_params=pltpu.CompilerParams(
            dimension_semantics=("parallel",),
        ),
    )(x, w.reshape(1, E))
    return out
